# Initial kernel scaffold; baseline (speedup 1.0000x reference)
import jax
import jax.numpy as jnp
from jax.experimental import pallas as pl


def kernel(h, edge_index):
    raise NotImplementedError("write your pallas kernel here")


# speedup vs baseline: 3.6887x; 3.6887x over previous
import functools

import jax
import jax.numpy as jnp
from jax import lax
from jax.experimental import pallas as pl
from jax.experimental.pallas import tpu as pltpu
from jax.experimental.pallas import tpu_sc as plsc

NC = 2
NS = 16
NW = NC * NS
LANES = 16
C = 128


def _dot_chunk(rows_s, rows_d, out_c, d_feat):
    n_seg = d_feat // LANES
    lanes = lax.iota(jnp.int32, LANES)

    gdn = lax.GatherDimensionNumbers(
        offset_dims=(), collapsed_slice_dims=(0,), start_index_map=(0,)
    )

    def perm(x, s):
        return lax.gather(
            x, (lanes ^ s)[:, None], gdn, (1,),
            mode=lax.GatherScatterMode.PROMISE_IN_BOUNDS,
        )

    def group_body(g, _):
        accs = []
        for i in range(LANES):
            e = g * LANES + i
            acc = rows_s[e, pl.ds(0, LANES)] * rows_d[e, pl.ds(0, LANES)]
            for j in range(1, n_seg):
                acc = acc + rows_s[e, pl.ds(j * LANES, LANES)] * rows_d[e, pl.ds(j * LANES, LANES)]
            accs.append(acc)
        for k in range(4):
            s = 1 << k
            mask = (lanes & s) == 0
            accs = [
                jnp.where(mask, a + perm(a, s), b + perm(b, s))
                for a, b in zip(accs[0::2], accs[1::2])
            ]
        out_c[pl.ds(g * LANES, LANES)] = accs[0]
        return 0

    lax.fori_loop(0, C // LANES, group_body, 0)


def _sc_dot(h, src, dst):
    n_nodes, d_feat = h.shape
    n_edges = src.shape[0]
    n_chunks = n_edges // C
    full_rounds = n_chunks // NW
    tail = n_chunks - full_rounds * NW

    mesh = plsc.VectorSubcoreMesh(
        core_axis_name="c", subcore_axis_name="s", num_cores=NC, num_subcores=NS
    )

    @functools.partial(
        pl.kernel,
        out_type=jax.ShapeDtypeStruct((n_edges,), jnp.float32),
        mesh=mesh,
        scratch_types=[
            pltpu.VMEM((2, C), jnp.int32),
            pltpu.VMEM((2, C), jnp.int32),
            pltpu.VMEM((2, C, d_feat), jnp.float32),
            pltpu.VMEM((2, C, d_feat), jnp.float32),
            pltpu.VMEM((2, C), jnp.float32),
            pltpu.SemaphoreType.DMA,
            pltpu.SemaphoreType.DMA,
            pltpu.SemaphoreType.DMA,
            pltpu.SemaphoreType.DMA,
        ],
    )
    def k(h_hbm, src_hbm, dst_hbm, out_hbm,
          idx_s, idx_d, rows_s, rows_d, out_c, sem_s0, sem_s1, sem_d0, sem_d1):
        wid = lax.axis_index("s") * NC + lax.axis_index("c")
        sems_s = (sem_s0, sem_s1)
        sems_d = (sem_d0, sem_d1)

        def chunk_off(i):
            return (i * NW + wid) * C

        def start(i, b):
            off = chunk_off(i)
            pltpu.sync_copy(src_hbm.at[pl.ds(off, C)], idx_s.at[b])
            pltpu.sync_copy(dst_hbm.at[pl.ds(off, C)], idx_d.at[b])
            pltpu.async_copy(h_hbm.at[idx_s.at[b]], rows_s.at[b], sems_s[b])
            pltpu.async_copy(h_hbm.at[idx_d.at[b]], rows_d.at[b], sems_d[b])

        def wait(b):
            pltpu.make_async_copy(h_hbm.at[idx_s.at[b]], rows_s.at[b], sems_s[b]).wait()
            pltpu.make_async_copy(h_hbm.at[idx_d.at[b]], rows_d.at[b], sems_d[b]).wait()

        def finish(i, b):
            wait(b)
            _dot_chunk(rows_s.at[b], rows_d.at[b], out_c.at[b], d_feat)
            pltpu.sync_copy(out_c.at[b], out_hbm.at[pl.ds(chunk_off(i), C)])

        start(0, 0)

        def loop_body(i, _):
            start(i + 1, 1)
            finish(i, 0)

            @pl.when(i + 2 < full_rounds)
            def _():
                start(i + 2, 0)

            finish(i + 1, 1)
            return 0

        lax.fori_loop(0, full_rounds // 2, lambda r, c: loop_body(r * 2, c), 0)

        if full_rounds % 2 == 1:
            finish(full_rounds - 1, 0)

        if tail:
            @pl.when(wid < tail)
            def _():
                off = (full_rounds * NW + wid) * C
                pltpu.sync_copy(src_hbm.at[pl.ds(off, C)], idx_s.at[0])
                pltpu.sync_copy(dst_hbm.at[pl.ds(off, C)], idx_d.at[0])
                pltpu.async_copy(h_hbm.at[idx_s.at[0]], rows_s.at[0], sem_s0)
                pltpu.async_copy(h_hbm.at[idx_d.at[0]], rows_d.at[0], sem_d0)
                pltpu.make_async_copy(h_hbm.at[idx_s.at[0]], rows_s.at[0], sem_s0).wait()
                pltpu.make_async_copy(h_hbm.at[idx_d.at[0]], rows_d.at[0], sem_d0).wait()
                _dot_chunk(rows_s.at[0], rows_d.at[0], out_c.at[0], d_feat)
                pltpu.sync_copy(out_c.at[0], out_hbm.at[pl.ds(off, C)])

    return k(h, src, dst)


@jax.jit
def kernel(h, edge_index):
    score = _sc_dot(h, edge_index[0], edge_index[1])
    return score[:, None]


# speedup vs baseline: 4.2953x; 1.1645x over previous
import functools

import jax
import jax.numpy as jnp
from jax import lax
from jax.experimental import pallas as pl
from jax.experimental.pallas import tpu as pltpu
from jax.experimental.pallas import tpu_sc as plsc

NC = 2
NS = 16
NW = NC * NS
LANES = 16
C = 128


def _dot_chunk(rows_s, rows_d, out_all, o_base, d_feat):
    n_seg = d_feat // LANES
    lanes = lax.iota(jnp.int32, LANES)
    gdn = lax.GatherDimensionNumbers(
        offset_dims=(), collapsed_slice_dims=(0,), start_index_map=(0,)
    )
    perm_idx = [((lanes ^ (1 << k))[:, None]) for k in range(4)]
    masks = [(lanes & (1 << k)) == 0 for k in range(4)]

    def perm(x, k):
        return lax.gather(
            x, perm_idx[k], gdn, (1,),
            mode=lax.GatherScatterMode.PROMISE_IN_BOUNDS,
        )

    def combine(a, b, k):
        return jnp.where(masks[k], a + perm(a, k), b + perm(b, k))

    def group_body(g, _):
        levels = [None] * 4
        node = None
        for i in range(LANES):
            e = g * LANES + i
            node = rows_s[e, pl.ds(0, LANES)] * rows_d[e, pl.ds(0, LANES)]
            for j in range(1, n_seg):
                node = node + rows_s[e, pl.ds(j * LANES, LANES)] * rows_d[e, pl.ds(j * LANES, LANES)]
            for k in range(4):
                if levels[k] is None:
                    levels[k] = node
                    node = None
                    break
                node = combine(levels[k], node, k)
                levels[k] = None
        off = pl.multiple_of(o_base + g * LANES, LANES)
        out_all[pl.ds(off, LANES)] = node
        return 0

    lax.fori_loop(0, C // LANES, group_body, 0)


def _sc_dot(h, src, dst):
    n_nodes, d_feat = h.shape
    n_edges = src.shape[0]
    n_chunks = n_edges // C
    base_rounds = n_chunks // NW
    tail = n_chunks - base_rounds * NW
    max_rounds = base_rounds + (1 if tail else 0)
    assert base_rounds % 2 == 0

    mesh = plsc.VectorSubcoreMesh(
        core_axis_name="c", subcore_axis_name="s", num_cores=NC, num_subcores=NS
    )

    @functools.partial(
        pl.kernel,
        out_type=jax.ShapeDtypeStruct((n_edges,), jnp.float32),
        mesh=mesh,
        scratch_types=[
            pltpu.VMEM((max_rounds * C,), jnp.int32),
            pltpu.VMEM((max_rounds * C,), jnp.int32),
            pltpu.VMEM((2, C, d_feat), jnp.float32),
            pltpu.VMEM((2, C, d_feat), jnp.float32),
            pltpu.VMEM((max_rounds * C,), jnp.float32),
            pltpu.SemaphoreType.DMA,
            pltpu.SemaphoreType.DMA,
            pltpu.SemaphoreType.DMA,
            pltpu.SemaphoreType.DMA,
        ],
    )
    def k(h_hbm, src_hbm, dst_hbm, out_hbm,
          idx_s, idx_d, rows_s, rows_d, out_all,
          sem_s0, sem_s1, sem_d0, sem_d1):
        wid = lax.axis_index("s") * NC + lax.axis_index("c")
        extra = wid < tail
        s_w = wid * base_rounds + jnp.minimum(wid, tail)
        e_w = pl.multiple_of(s_w * C, C)
        n_base = base_rounds * C
        sems_s = (sem_s0, sem_s1)
        sems_d = (sem_d0, sem_d1)

        pltpu.sync_copy(src_hbm.at[pl.ds(e_w, n_base)], idx_s.at[pl.ds(0, n_base)])
        pltpu.sync_copy(dst_hbm.at[pl.ds(e_w, n_base)], idx_d.at[pl.ds(0, n_base)])
        if tail:
            @pl.when(extra)
            def _():
                pltpu.sync_copy(src_hbm.at[pl.ds(e_w + n_base, C)],
                                idx_s.at[pl.ds(n_base, C)])
                pltpu.sync_copy(dst_hbm.at[pl.ds(e_w + n_base, C)],
                                idx_d.at[pl.ds(n_base, C)])

        def start(i, b):
            off = pl.multiple_of(i * C, C)
            pltpu.async_copy(h_hbm.at[idx_s.at[pl.ds(off, C)]], rows_s.at[b], sems_s[b])
            pltpu.async_copy(h_hbm.at[idx_d.at[pl.ds(off, C)]], rows_d.at[b], sems_d[b])

        def finish(i, b):
            off = pl.multiple_of(i * C, C)
            pltpu.make_async_copy(h_hbm.at[idx_s.at[pl.ds(off, C)]], rows_s.at[b], sems_s[b]).wait()
            pltpu.make_async_copy(h_hbm.at[idx_d.at[pl.ds(off, C)]], rows_d.at[b], sems_d[b]).wait()
            _dot_chunk(rows_s.at[b], rows_d.at[b], out_all, off, d_feat)

        start(0, 0)

        def loop_body(i, _):
            start(i + 1, 1)
            finish(i, 0)

            @pl.when(i + 2 < base_rounds)
            def _():
                start(i + 2, 0)

            finish(i + 1, 1)
            return 0

        lax.fori_loop(0, base_rounds // 2, lambda r, c: loop_body(r * 2, c), 0)

        if tail:
            @pl.when(extra)
            def _():
                start(base_rounds, 0)
                finish(base_rounds, 0)

        pltpu.sync_copy(out_all.at[pl.ds(0, n_base)], out_hbm.at[pl.ds(e_w, n_base)])
        if tail:
            @pl.when(extra)
            def _():
                pltpu.sync_copy(out_all.at[pl.ds(n_base, C)],
                                out_hbm.at[pl.ds(e_w + n_base, C)])

    return k(h, src, dst)


@jax.jit
def kernel(h, edge_index):
    score = _sc_dot(h, edge_index[0], edge_index[1])
    return score[:, None]


# speedup vs baseline: 8.9481x; 2.0832x over previous
import functools

import jax
import jax.numpy as jnp
from jax import lax
from jax.experimental import pallas as pl
from jax.experimental.pallas import tpu as pltpu
from jax.experimental.pallas import tpu_sc as plsc

NC = 2
NS = 16
NW = NC * NS
LANES = 16
C = 128
NBUF = 3


def _sqnorm_tc(h):
    n_nodes, d_feat = h.shape
    blk = 2000
    assert n_nodes % blk == 0

    def body(h_ref, o_ref):
        x = h_ref[...]
        o_ref[...] = jnp.sum(x * x, axis=1, keepdims=True)

    return pl.pallas_call(
        body,
        grid=(n_nodes // blk,),
        in_specs=[pl.BlockSpec((blk, d_feat), lambda i: (i, 0))],
        out_specs=pl.BlockSpec((blk, 1), lambda i: (i, 0)),
        out_shape=jax.ShapeDtypeStruct((n_nodes, 1), jnp.float32),
    )(h)


def _dot_chunk(rows_w, sq, idx_s, idx_d, off, out_c, d_feat):
    n_seg = d_feat // LANES
    lanes = lax.iota(jnp.int32, LANES)
    gdn = lax.GatherDimensionNumbers(
        offset_dims=(), collapsed_slice_dims=(0,), start_index_map=(0,)
    )
    perm_idx = [((lanes ^ (1 << k))[:, None]) for k in range(4)]
    masks = [(lanes & (1 << k)) == 0 for k in range(4)]

    def perm(x, k):
        return lax.gather(
            x, perm_idx[k], gdn, (1,),
            mode=lax.GatherScatterMode.PROMISE_IN_BOUNDS,
        )

    def combine(a, b, k):
        return jnp.where(masks[k], a + perm(a, k), b + perm(b, k))

    def group_body(g, _):
        levels = [None] * 4
        node = None
        for i in range(LANES):
            e = g * LANES + i
            node = None
            for j in range(n_seg):
                w = rows_w[e, pl.ds(j * LANES, LANES)]
                t = w * w
                node = t if node is None else node + t
            for k in range(4):
                if levels[k] is None:
                    levels[k] = node
                    node = None
                    break
                node = combine(levels[k], node, k)
                levels[k] = None
        ebase = pl.multiple_of(off + g * LANES, LANES)
        su = plsc.load_gather(sq, [idx_s[pl.ds(ebase, LANES)]])
        sv = plsc.load_gather(sq, [idx_d[pl.ds(ebase, LANES)]])
        out_c[pl.ds(g * LANES, LANES)] = (node - su - sv) * 0.5
        return 0

    lax.fori_loop(0, C // LANES, group_body, 0)


def _sc_dot(h, sq, src, dst):
    n_nodes, d_feat = h.shape
    n_edges = src.shape[0]
    n_chunks = n_edges // C
    base_rounds = n_chunks // NW
    tail = n_chunks - base_rounds * NW
    assert base_rounds % NBUF == 0 and base_rounds >= 2 * NBUF

    mesh = plsc.VectorSubcoreMesh(
        core_axis_name="c", subcore_axis_name="s", num_cores=NC, num_subcores=NS
    )

    @functools.partial(
        pl.kernel,
        out_type=jax.ShapeDtypeStruct((n_edges,), jnp.float32),
        mesh=mesh,
        compiler_params=pltpu.CompilerParams(needs_layout_passes=False),
        scratch_types=[
            pltpu.VMEM(((base_rounds + (1 if tail else 0)) * C,), jnp.int32),
            pltpu.VMEM(((base_rounds + (1 if tail else 0)) * C,), jnp.int32),
            pltpu.VMEM((NBUF, C, d_feat), jnp.float32),
            pltpu.VMEM((n_nodes,), jnp.float32),
            pltpu.VMEM((2, C), jnp.float32),
            pltpu.SemaphoreType.DMA,
            pltpu.SemaphoreType.DMA,
            pltpu.SemaphoreType.DMA,
            pltpu.SemaphoreType.DMA,
            pltpu.SemaphoreType.DMA,
        ],
    )
    def k(h_hbm, sq_hbm, src_hbm, dst_hbm, out_hbm,
          idx_s, idx_d, rows_w, sq_v, out_ring,
          sem_w0, sem_w1, sem_w2, sem_o0, sem_o1):
        wid = lax.axis_index("s") * NC + lax.axis_index("c")
        extra = wid < tail
        n_w = base_rounds + extra.astype(jnp.int32)
        s_w = wid * base_rounds + jnp.minimum(wid, tail)
        e_w = pl.multiple_of(s_w * C, C)
        n_base = base_rounds * C
        sems_w = (sem_w0, sem_w1, sem_w2)
        sems_o = (sem_o0, sem_o1)

        pltpu.sync_copy(sq_hbm.at[pl.ds(0, n_nodes)], sq_v)
        pltpu.sync_copy(src_hbm.at[pl.ds(e_w, n_base)], idx_s.at[pl.ds(0, n_base)])
        pltpu.sync_copy(dst_hbm.at[pl.ds(e_w, n_base)], idx_d.at[pl.ds(0, n_base)])
        if tail:
            @pl.when(extra)
            def _():
                pltpu.sync_copy(src_hbm.at[pl.ds(e_w + n_base, C)],
                                idx_s.at[pl.ds(n_base, C)])
                pltpu.sync_copy(dst_hbm.at[pl.ds(e_w + n_base, C)],
                                idx_d.at[pl.ds(n_base, C)])

        def start_p1(i, b):
            off = pl.multiple_of(i * C, C)
            pltpu.async_copy(h_hbm.at[idx_s.at[pl.ds(off, C)]], rows_w.at[b], sems_w[b])

        def start_p2(i, b):
            off = pl.multiple_of(i * C, C)
            pltpu.make_async_copy(h_hbm.at[idx_s.at[pl.ds(off, C)]], rows_w.at[b], sems_w[b]).wait()
            pltpu.async_copy(h_hbm.at[idx_d.at[pl.ds(off, C)]], rows_w.at[b], sems_w[b], add=True)

        def compute(i, b, ob):
            off = pl.multiple_of(i * C, C)
            pltpu.make_async_copy(h_hbm.at[idx_d.at[pl.ds(off, C)]], rows_w.at[b], sems_w[b]).wait()

            @pl.when(i >= 2)
            def _():
                pltpu.make_async_copy(out_ring.at[ob], out_hbm.at[pl.ds(e_w, C)], sems_o[ob]).wait()

            _dot_chunk(rows_w.at[b], sq_v, idx_s, idx_d, off, out_ring.at[ob], d_feat)
            pltpu.async_copy(out_ring.at[ob], out_hbm.at[pl.ds(e_w + off, C)], sems_o[ob])

        start_p1(0, 0)
        start_p1(1, 1)
        start_p2(0, 0)

        def step(i, b, ob):
            @pl.when(i + 2 < n_w)
            def _():
                start_p1(i + 2, (b + 2) % NBUF)

            @pl.when(i + 1 < n_w)
            def _():
                start_p2(i + 1, (b + 1) % NBUF)

            compute(i, b, ob)

        def loop_body(r, _):
            i = r * NBUF
            for u in range(NBUF):
                step(i + u, u, 0 if u % 2 == 0 else 1)
            return 0

        lax.fori_loop(0, base_rounds // NBUF, loop_body, 0)

        if tail:
            @pl.when(extra)
            def _():
                compute(base_rounds, base_rounds % NBUF, base_rounds % 2)

        pltpu.make_async_copy(out_ring.at[0], out_hbm.at[pl.ds(e_w, C)], sem_o0).wait()
        pltpu.make_async_copy(out_ring.at[1], out_hbm.at[pl.ds(e_w, C)], sem_o1).wait()

    return k(h, sq, src, dst)


@jax.jit
def kernel(h, edge_index):
    sq = _sqnorm_tc(h).reshape(-1)
    score = _sc_dot(h, sq, edge_index[0], edge_index[1])
    return score[:, None]
